# Initial kernel scaffold; baseline (speedup 1.0000x reference)
import jax
import jax.numpy as jnp
from jax.experimental import pallas as pl


def kernel(x, Wq, Wk, Wv, Wb, Wa, A_log, dt_bias, Wg, norm_weight, Wo):
    raise NotImplementedError("write your pallas kernel here")


# speedup vs baseline: 11.5739x; 11.5739x over previous
import jax
import jax.numpy as jnp
from jax.experimental import pallas as pl
from jax.experimental.pallas import tpu as pltpu

B, T, D = 1, 2048, 1024
H, HD, NH = 16, 64, 2
L = T * NH
EPS = 1e-5
SCALE = HD ** -0.5

PCOLS_RAW = H * HD + 2 * (NH * H * HD) + D + NH * H + H
PCOLS = 6272
PR_BM, PR_BN = 512, 896

CHUNK = 64
NC = L // CHUNK
GB = 2
HG = H // GB

OB_M = 512


def _dot(a, b):
    return jax.lax.dot_general(a, b, (((1,), (0,)), ((), ())),
                               preferred_element_type=jnp.float32)


def _dot_nt(a, b):
    return jax.lax.dot_general(a, b, (((1,), (1,)), ((), ())),
                               preferred_element_type=jnp.float32)


def _dot_tn(a, b):
    return jax.lax.dot_general(a, b, (((0,), (0,)), ((), ())),
                               preferred_element_type=jnp.float32)


def _proj_body(x_ref, w_ref, o_ref):
    o_ref[...] = _dot(x_ref[...], w_ref[...])


def _delta_body(k_ref, v_ref, q_ref, bc_ref, gc_ref, gr_ref, o_ref, s_ref):
    c = pl.program_id(1)

    @pl.when(c == 0)
    def _():
        s_ref[...] = jnp.zeros_like(s_ref)

    C = CHUNK
    row = jax.lax.broadcasted_iota(jnp.int32, (C, C), 0)
    col = jax.lax.broadcasted_iota(jnp.int32, (C, C), 1)
    incl = row >= col
    strict = row > col
    lec = row <= col

    for i in range(HG):
        k = k_ref[i]
        v = v_ref[i]
        q = q_ref[i]
        bcol = bc_ref[i, 0]
        gcol = gc_ref[i, 0]
        grow = gr_ref[i, 0]
        S = s_ref[i]

        kn = k * jax.lax.rsqrt(jnp.sum(k * k, axis=1, keepdims=True) + 1e-6)
        qn = q * jax.lax.rsqrt(jnp.sum(q * q, axis=1, keepdims=True) + 1e-6) * SCALE

        Grow = jnp.sum(jnp.where(incl, jnp.broadcast_to(grow, (C, C)), 0.0),
                       axis=1, keepdims=True)
        Gcol = jnp.sum(jnp.where(lec, jnp.broadcast_to(gcol, (C, C)), 0.0),
                       axis=0, keepdims=True)
        eG = jnp.exp(Grow)
        Glast = jnp.sum(grow)
        Dfull = jnp.exp(jnp.where(incl, Grow - Gcol, -1e30))
        Dincl = jnp.where(incl, Dfull, 0.0)
        Dstrict = jnp.where(strict, Dfull, 0.0)

        Nm = _dot_nt(kn * bcol, kn) * (-Dstrict)
        rhs = bcol * (v - eG * _dot(kn, S))

        t = rhs
        Np = Nm
        for j in range(6):
            t = t + _dot(Np, t)
            if j < 5:
                Np = _dot(Np, Np)

        attn = _dot_nt(qn, kn) * Dincl
        o_ref[i] = eG * _dot(qn, S) + _dot(attn, t)
        s_ref[i] = jnp.exp(Glast) * S + _dot_tn(kn * jnp.exp(Glast - Grow), t)


def _out_body(o_ref, g_ref, w_ref, nw_ref, y_ref):
    parts = []
    for h in range(H):
        o = o_ref[h]
        on = o * jax.lax.rsqrt(jnp.mean(o * o, axis=1, keepdims=True) + EPS)
        on = on * nw_ref[...]
        gg = g_ref[:, h * HD:(h + 1) * HD]
        parts.append(on * (gg * jax.nn.sigmoid(gg)))
    y = jnp.concatenate(parts, axis=1)
    y_ref[...] = _dot(y, w_ref[...])


def kernel(x, Wq, Wk, Wv, Wb, Wa, A_log, dt_bias, Wg, norm_weight, Wo):
    x2 = x.reshape(T, D)
    Wall = jnp.concatenate([Wq, Wk, Wv, Wg, Wb, Wa], axis=1)
    Wall = jnp.pad(Wall, ((0, 0), (0, PCOLS - PCOLS_RAW)))

    P = pl.pallas_call(
        _proj_body,
        out_shape=jax.ShapeDtypeStruct((T, PCOLS), jnp.float32),
        grid=(T // PR_BM, PCOLS // PR_BN),
        in_specs=[pl.BlockSpec((PR_BM, D), lambda i, j: (i, 0)),
                  pl.BlockSpec((D, PR_BN), lambda i, j: (0, j))],
        out_specs=pl.BlockSpec((PR_BM, PR_BN), lambda i, j: (i, j)),
        compiler_params=pltpu.CompilerParams(
            dimension_semantics=("parallel", "arbitrary")),
        name="deltanet_proj",
    )(x2, Wall)

    qraw = P[:, 0:1024]
    kraw = P[:, 1024:3072]
    vraw = P[:, 3072:5120]
    graw = P[:, 5120:6144]
    braw = P[:, 6144:6176]
    araw = P[:, 6176:6192]

    qstep = jnp.repeat(qraw.reshape(T, H, HD).transpose(1, 0, 2), NH, axis=1)
    kstep = kraw.reshape(T, NH, H, HD).transpose(2, 0, 1, 3).reshape(H, L, HD)
    vstep = vraw.reshape(T, NH, H, HD).transpose(2, 0, 1, 3).reshape(H, L, HD)

    beta = 2.0 * jax.nn.sigmoid(braw)
    beta = beta.reshape(T, NH, H).transpose(2, 0, 1).reshape(H, L)
    g_tok = -jnp.exp(A_log)[None, :] * jax.nn.softplus(araw + dt_bias[None, :])
    gstep = jnp.stack([g_tok.T, jnp.zeros((H, T), jnp.float32)], axis=2).reshape(H, L)

    b_col = beta.reshape(H, NC, CHUNK, 1)
    g_col = gstep.reshape(H, NC, CHUNK, 1)
    g_row = gstep.reshape(H, NC, 1, CHUNK)

    Ofull = pl.pallas_call(
        _delta_body,
        out_shape=jax.ShapeDtypeStruct((H, L, HD), jnp.float32),
        grid=(GB, NC),
        in_specs=[
            pl.BlockSpec((HG, CHUNK, HD), lambda g, c: (g, c, 0)),
            pl.BlockSpec((HG, CHUNK, HD), lambda g, c: (g, c, 0)),
            pl.BlockSpec((HG, CHUNK, HD), lambda g, c: (g, c, 0)),
            pl.BlockSpec((HG, 1, CHUNK, 1), lambda g, c: (g, c, 0, 0)),
            pl.BlockSpec((HG, 1, CHUNK, 1), lambda g, c: (g, c, 0, 0)),
            pl.BlockSpec((HG, 1, 1, CHUNK), lambda g, c: (g, c, 0, 0)),
        ],
        out_specs=pl.BlockSpec((HG, CHUNK, HD), lambda g, c: (g, c, 0)),
        scratch_shapes=[pltpu.VMEM((HG, HD, HD), jnp.float32)],
        compiler_params=pltpu.CompilerParams(
            dimension_semantics=("parallel", "arbitrary")),
        name="deltanet_chunk",
    )(kstep, vstep, qstep, b_col, g_col, g_row)

    O_tok = Ofull[:, 1::2, :]

    y = pl.pallas_call(
        _out_body,
        out_shape=jax.ShapeDtypeStruct((T, D), jnp.float32),
        grid=(T // OB_M,),
        in_specs=[
            pl.BlockSpec((H, OB_M, HD), lambda i: (0, i, 0)),
            pl.BlockSpec((OB_M, H * HD), lambda i: (i, 0)),
            pl.BlockSpec((H * HD, D), lambda i: (0, 0)),
            pl.BlockSpec((1, HD), lambda i: (0, 0)),
        ],
        out_specs=pl.BlockSpec((OB_M, D), lambda i: (i, 0)),
        compiler_params=pltpu.CompilerParams(
            dimension_semantics=("parallel",)),
        name="deltanet_out",
    )(O_tok, graw, Wo, norm_weight.reshape(1, HD))

    return y.reshape(B, T, D)


# speedup vs baseline: 14.3781x; 1.2423x over previous
import jax
import jax.numpy as jnp
from jax.experimental import pallas as pl
from jax.experimental.pallas import tpu as pltpu

B, T, D = 1, 2048, 1024
H, HD, NH = 16, 64, 2
L = T * NH
EPS = 1e-5
SCALE = HD ** -0.5

PCOLS_RAW = H * HD + 2 * (NH * H * HD) + D + NH * H + H
PCOLS = 6272
PR_BM, PR_BN = 512, 896

CT = 32
CHUNK = NH * CT
NC = T // CT

OB_M = 512

QO, KO, VO, GO, BO, AO = 0, 1024, 3072, 5120, 6144, 6176


def _dot(a, b):
    return jax.lax.dot_general(a, b, (((1,), (0,)), ((), ())),
                               preferred_element_type=jnp.float32)


def _dot_nt(a, b):
    return jax.lax.dot_general(a, b, (((1,), (1,)), ((), ())),
                               preferred_element_type=jnp.float32)


def _dot_tn(a, b):
    return jax.lax.dot_general(a, b, (((0,), (0,)), ((), ())),
                               preferred_element_type=jnp.float32)


def _proj_body(x_ref, w_ref, o_ref):
    o_ref[...] = _dot(x_ref[...], w_ref[...])


def _delta_body(p_ref, bc_ref, gc_ref, gr_ref, o_ref, s_ref):
    c = pl.program_id(0)

    @pl.when(c == 0)
    def _():
        s_ref[...] = jnp.zeros_like(s_ref)

    C = CHUNK
    r = jax.lax.broadcasted_iota(jnp.int32, (C, C), 0)
    q_ = jax.lax.broadcasted_iota(jnp.int32, (C, C), 1)
    rstep = jnp.where(r < CT, 2 * r, 2 * (r - CT) + 1)
    cstep = jnp.where(q_ < CT, 2 * q_, 2 * (q_ - CT) + 1)
    incl = rstep >= cstep
    strict = rstep > cstep

    rt = jax.lax.broadcasted_iota(jnp.int32, (CT, CT), 0)
    ct_ = jax.lax.broadcasted_iota(jnp.int32, (CT, CT), 1)
    le_col = rt <= ct_
    ge_col = ct_ <= rt

    bblk = bc_ref[0]
    gcb = gc_ref[0]
    grb = gr_ref[0]

    R = range(H)
    kp, vp, qn, bcol, eG, eGlast, eCI, Dincl, Dstrict, S, eGo = \
        [], [], [], [], [], [], [], [], [], [], []
    for i in R:
        qr = p_ref[:, QO + i * HD:QO + (i + 1) * HD]
        k0 = p_ref[:, KO + i * HD:KO + (i + 1) * HD]
        k1 = p_ref[:, KO + 1024 + i * HD:KO + 1024 + (i + 1) * HD]
        v0 = p_ref[:, VO + i * HD:VO + (i + 1) * HD]
        v1 = p_ref[:, VO + 1024 + i * HD:VO + 1024 + (i + 1) * HD]
        S.append(s_ref[i])

        k = jnp.concatenate([k0, k1], axis=0)
        kp.append(k * jax.lax.rsqrt(jnp.sum(k * k, axis=1, keepdims=True) + 1e-6))
        vp.append(jnp.concatenate([v0, v1], axis=0))
        qn.append(qr * jax.lax.rsqrt(jnp.sum(qr * qr, axis=1, keepdims=True) + 1e-6)
                  * SCALE)
        bcol.append(jnp.concatenate([bblk[:, i:i + 1], bblk[:, H + i:H + i + 1]],
                                    axis=0))

        gcol = gcb[:, i:i + 1]
        grow = grb[i:i + 1, :]
        Gtc = jnp.sum(jnp.where(ge_col, jnp.broadcast_to(grow, (CT, CT)), 0.0),
                      axis=1, keepdims=True)
        Gtr = jnp.sum(jnp.where(le_col, jnp.broadcast_to(gcol, (CT, CT)), 0.0),
                      axis=0, keepdims=True)
        Growp = jnp.concatenate([Gtc, Gtc], axis=0)
        Gcolp = jnp.concatenate([Gtr, Gtr], axis=1)
        Glast = jnp.sum(grow)
        eG.append(jnp.exp(Growp))
        eGo.append(jnp.exp(Gtc))
        eGlast.append(jnp.exp(Glast))
        eCI.append(jnp.exp(Glast - Growp))
        Dfull = jnp.exp(jnp.where(incl, Growp - Gcolp, -1e30))
        Dincl.append(jnp.where(incl, Dfull, 0.0)[CT:, :])
        Dstrict.append(jnp.where(strict, Dfull, 0.0))

    Np = [_dot_nt(kp[i] * bcol[i], kp[i]) * (-Dstrict[i]) for i in R]
    pred = [_dot(kp[i], S[i]) for i in R]
    attn = [_dot_nt(qn[i], kp[i]) * Dincl[i] for i in R]
    oq = [_dot(qn[i], S[i]) for i in R]
    t = [bcol[i] * (vp[i] - eG[i] * pred[i]) for i in R]

    for j in range(6):
        t = [t[i] + _dot(Np[i], t[i]) for i in R]
        if j < 5:
            Np = [_dot(Np[i], Np[i]) for i in R]

    for i in R:
        o_ref[i] = eGo[i] * oq[i] + _dot(attn[i], t[i])
    for i in R:
        s_ref[i] = eGlast[i] * S[i] + _dot_tn(kp[i] * eCI[i], t[i])


def _out_body(o_ref, g_ref, w_ref, nw_ref, y_ref):
    parts = []
    for h in range(H):
        o = o_ref[h]
        on = o * jax.lax.rsqrt(jnp.mean(o * o, axis=1, keepdims=True) + EPS)
        on = on * nw_ref[...]
        gg = g_ref[:, h * HD:(h + 1) * HD]
        parts.append(on * (gg * jax.nn.sigmoid(gg)))
    y = jnp.concatenate(parts, axis=1)
    y_ref[...] = _dot(y, w_ref[...])


def kernel(x, Wq, Wk, Wv, Wb, Wa, A_log, dt_bias, Wg, norm_weight, Wo):
    x2 = x.reshape(T, D)
    Wall = jnp.concatenate([Wq, Wk, Wv, Wg, Wb, Wa], axis=1)
    Wall = jnp.pad(Wall, ((0, 0), (0, PCOLS - PCOLS_RAW)))

    P = pl.pallas_call(
        _proj_body,
        out_shape=jax.ShapeDtypeStruct((T, PCOLS), jnp.float32),
        grid=(T // PR_BM, PCOLS // PR_BN),
        in_specs=[pl.BlockSpec((PR_BM, D), lambda i, j: (i, 0)),
                  pl.BlockSpec((D, PR_BN), lambda i, j: (0, j))],
        out_specs=pl.BlockSpec((PR_BM, PR_BN), lambda i, j: (i, j)),
        compiler_params=pltpu.CompilerParams(
            dimension_semantics=("arbitrary", "arbitrary")),
        name="deltanet_proj",
    )(x2, Wall)

    braw = P[:, BO:BO + 2 * H]
    araw = P[:, AO:AO + H]
    bact = (2.0 * jax.nn.sigmoid(braw)).reshape(NC, CT, 2 * H)
    g_tok = -jnp.exp(A_log)[None, :] * jax.nn.softplus(araw + dt_bias[None, :])
    g_colarr = g_tok.reshape(NC, CT, H)
    g_rowarr = g_colarr.transpose(0, 2, 1)

    O_tok = pl.pallas_call(
        _delta_body,
        out_shape=jax.ShapeDtypeStruct((H, T, HD), jnp.float32),
        grid=(NC,),
        in_specs=[
            pl.BlockSpec((CT, GO), lambda c: (c, 0)),
            pl.BlockSpec((1, CT, 2 * H), lambda c: (c, 0, 0)),
            pl.BlockSpec((1, CT, H), lambda c: (c, 0, 0)),
            pl.BlockSpec((1, H, CT), lambda c: (c, 0, 0)),
        ],
        out_specs=pl.BlockSpec((H, CT, HD), lambda c: (0, c, 0)),
        scratch_shapes=[pltpu.VMEM((H, HD, HD), jnp.float32)],
        compiler_params=pltpu.CompilerParams(
            dimension_semantics=("arbitrary",)),
        name="deltanet_chunk",
    )(P, bact, g_colarr, g_rowarr)

    y = pl.pallas_call(
        _out_body,
        out_shape=jax.ShapeDtypeStruct((T, D), jnp.float32),
        grid=(T // OB_M,),
        in_specs=[
            pl.BlockSpec((H, OB_M, HD), lambda i: (0, i, 0)),
            pl.BlockSpec((OB_M, H * HD), lambda i: (i, 5)),
            pl.BlockSpec((H * HD, D), lambda i: (0, 0)),
            pl.BlockSpec((1, HD), lambda i: (0, 0)),
        ],
        out_specs=pl.BlockSpec((OB_M, D), lambda i: (i, 0)),
        compiler_params=pltpu.CompilerParams(
            dimension_semantics=("arbitrary",)),
        name="deltanet_out",
    )(O_tok, P, Wo, norm_weight.reshape(1, HD))

    return y.reshape(B, T, D)


# speedup vs baseline: 33.6335x; 2.3392x over previous
import jax
import jax.numpy as jnp
from jax.experimental import pallas as pl
from jax.experimental.pallas import tpu as pltpu

B, T, D = 1, 2048, 1024
H, HD, NH = 16, 64, 2
L = T * NH
EPS = 1e-5
SCALE = HD ** -0.5

PCOLS_RAW = H * HD + 2 * (NH * H * HD) + D + NH * H + H
PCOLS = 6272
PR_BM, PR_BN = 512, 896

CT = 32
CHUNK = NH * CT
NC = T // CT

OB_M = 512

GO = 5120


def _dot(a, b):
    return jax.lax.dot_general(a, b, (((1,), (0,)), ((), ())),
                               preferred_element_type=jnp.float32)


def _dot_nt(a, b):
    return jax.lax.dot_general(a, b, (((1,), (1,)), ((), ())),
                               preferred_element_type=jnp.float32)


def _dot_tn(a, b):
    return jax.lax.dot_general(a, b, (((0,), (0,)), ((), ())),
                               preferred_element_type=jnp.float32)


def _proj_body(x_ref, w_ref, o_ref):
    o_ref[...] = _dot(x_ref[...], w_ref[...])


def _delta_body(k_ref, v_ref, q_ref, bc_ref, gc_ref, gr_ref, o_ref, s_ref):
    c = pl.program_id(0)

    @pl.when(c == 0)
    def _():
        s_ref[...] = jnp.zeros_like(s_ref)

    C = CHUNK
    row = jax.lax.broadcasted_iota(jnp.int32, (C, C), 0)
    col = jax.lax.broadcasted_iota(jnp.int32, (C, C), 1)
    incl = row >= col
    strict = row > col
    lec = row <= col
    rtok = jax.lax.broadcasted_iota(jnp.int32, (CT, C), 0)
    codd = jax.lax.broadcasted_iota(jnp.int32, (CT, C), 1)
    incl_odd = codd <= 2 * rtok + 1

    R = range(H)
    kn, qn, v, bcol, eG, eGlast, eCI, Dincl, Dstrict, S, eGo = \
        [], [], [], [], [], [], [], [], [], [], []
    for i in R:
        k = k_ref[i]
        q = q_ref[i]
        v.append(v_ref[i])
        bcol.append(bc_ref[i, 0])
        gcol = gc_ref[i, 0]
        grow = gr_ref[i, 0]
        S.append(s_ref[i])

        kn.append(k * jax.lax.rsqrt(jnp.sum(k * k, axis=1, keepdims=True) + 1e-6))
        qn.append(q * jax.lax.rsqrt(jnp.sum(q * q, axis=1, keepdims=True) + 1e-6)
                  * SCALE)

        Grow = jnp.sum(jnp.where(incl, jnp.broadcast_to(grow, (C, C)), 0.0),
                       axis=1, keepdims=True)
        Gcol = jnp.sum(jnp.where(lec, jnp.broadcast_to(gcol, (C, C)), 0.0),
                       axis=0, keepdims=True)
        Godd = jnp.sum(jnp.where(incl_odd, jnp.broadcast_to(grow, (CT, C)), 0.0),
                       axis=1, keepdims=True)
        eG.append(jnp.exp(Grow))
        eGo.append(jnp.exp(Godd))
        Glast = jnp.sum(grow)
        eGlast.append(jnp.exp(Glast))
        eCI.append(jnp.exp(Glast - Grow))
        Dfull = jnp.exp(jnp.where(incl, Grow - Gcol, -1e30))
        Dstrict.append(jnp.where(strict, Dfull, 0.0))
        Dodd = jnp.exp(jnp.where(incl_odd, Godd - Gcol, -1e30))
        Dincl.append(jnp.where(incl_odd, Dodd, 0.0))

    Np = [_dot_nt(kn[i] * bcol[i], kn[i]) * (-Dstrict[i]) for i in R]
    pred = [_dot(kn[i], S[i]) for i in R]
    attn = [_dot_nt(qn[i], kn[i]) * Dincl[i] for i in R]
    oq = [_dot(qn[i], S[i]) for i in R]
    t = [bcol[i] * (v[i] - eG[i] * pred[i]) for i in R]

    for j in range(6):
        t = [t[i] + _dot(Np[i], t[i]) for i in R]
        if j < 5:
            Np = [_dot(Np[i], Np[i]) for i in R]

    for i in R:
        o_ref[i] = eGo[i] * oq[i] + _dot(attn[i], t[i])
    for i in R:
        s_ref[i] = eGlast[i] * S[i] + _dot_tn(kn[i] * eCI[i], t[i])


def _out_body(o_ref, g_ref, w_ref, nw_ref, y_ref):
    parts = []
    for h in range(H):
        o = o_ref[h]
        on = o * jax.lax.rsqrt(jnp.mean(o * o, axis=1, keepdims=True) + EPS)
        on = on * nw_ref[...]
        gg = g_ref[:, h * HD:(h + 1) * HD]
        parts.append(on * (gg * jax.nn.sigmoid(gg)))
    y = jnp.concatenate(parts, axis=1)
    y_ref[...] = _dot(y, w_ref[...])


def kernel(x, Wq, Wk, Wv, Wb, Wa, A_log, dt_bias, Wg, norm_weight, Wo):
    x2 = x.reshape(T, D)
    Wall = jnp.concatenate([Wq, Wk, Wv, Wg, Wb, Wa], axis=1)
    Wall = jnp.pad(Wall, ((0, 0), (0, PCOLS - PCOLS_RAW)))

    P = pl.pallas_call(
        _proj_body,
        out_shape=jax.ShapeDtypeStruct((T, PCOLS), jnp.float32),
        grid=(T // PR_BM, PCOLS // PR_BN),
        in_specs=[pl.BlockSpec((PR_BM, D), lambda i, j: (i, 0)),
                  pl.BlockSpec((D, PR_BN), lambda i, j: (0, j))],
        out_specs=pl.BlockSpec((PR_BM, PR_BN), lambda i, j: (i, j)),
        compiler_params=pltpu.CompilerParams(
            dimension_semantics=("arbitrary", "arbitrary")),
        name="deltanet_proj",
    )(x2, Wall)

    qraw = P[:, 0:1024]
    kraw = P[:, 1024:3072]
    vraw = P[:, 3072:5120]
    braw = P[:, 6144:6176]
    araw = P[:, 6176:6192]

    qarr = qraw.reshape(T, H, HD).transpose(1, 0, 2)
    kstep = kraw.reshape(T, NH, H, HD).transpose(2, 0, 1, 3).reshape(H, L, HD)
    vstep = vraw.reshape(T, NH, H, HD).transpose(2, 0, 1, 3).reshape(H, L, HD)

    beta = 2.0 * jax.nn.sigmoid(braw)
    beta = beta.reshape(T, NH, H).transpose(2, 0, 1).reshape(H, L)
    g_tok = -jnp.exp(A_log)[None, :] * jax.nn.softplus(araw + dt_bias[None, :])
    gstep = jnp.stack([g_tok.T, jnp.zeros((H, T), jnp.float32)], axis=2).reshape(H, L)

    b_col = beta.reshape(H, NC, CHUNK, 1)
    g_col = gstep.reshape(H, NC, CHUNK, 1)
    g_row = gstep.reshape(H, NC, 1, CHUNK)

    O_tok = pl.pallas_call(
        _delta_body,
        out_shape=jax.ShapeDtypeStruct((H, T, HD), jnp.float32),
        grid=(NC,),
        in_specs=[
            pl.BlockSpec((H, CHUNK, HD), lambda c: (0, c, 0)),
            pl.BlockSpec((H, CHUNK, HD), lambda c: (0, c, 0)),
            pl.BlockSpec((H, CT, HD), lambda c: (0, c, 0)),
            pl.BlockSpec((H, 1, CHUNK, 1), lambda c: (0, c, 0, 0)),
            pl.BlockSpec((H, 1, CHUNK, 1), lambda c: (0, c, 0, 0)),
            pl.BlockSpec((H, 1, 1, CHUNK), lambda c: (0, c, 0, 0)),
        ],
        out_specs=pl.BlockSpec((H, CT, HD), lambda c: (0, c, 0)),
        scratch_shapes=[pltpu.VMEM((H, HD, HD), jnp.float32)],
        compiler_params=pltpu.CompilerParams(
            dimension_semantics=("arbitrary",)),
        name="deltanet_chunk",
    )(kstep, vstep, qarr, b_col, g_col, g_row)

    y = pl.pallas_call(
        _out_body,
        out_shape=jax.ShapeDtypeStruct((T, D), jnp.float32),
        grid=(T // OB_M,),
        in_specs=[
            pl.BlockSpec((H, OB_M, HD), lambda i: (0, i, 0)),
            pl.BlockSpec((OB_M, H * HD), lambda i: (i, 5)),
            pl.BlockSpec((H * HD, D), lambda i: (0, 0)),
            pl.BlockSpec((1, HD), lambda i: (0, 0)),
        ],
        out_specs=pl.BlockSpec((OB_M, D), lambda i: (i, 0)),
        compiler_params=pltpu.CompilerParams(
            dimension_semantics=("arbitrary",)),
        name="deltanet_out",
    )(O_tok, P, Wo, norm_weight.reshape(1, HD))

    return y.reshape(B, T, D)


# speedup vs baseline: 37.1965x; 1.1059x over previous
import jax
import jax.numpy as jnp
from jax.experimental import pallas as pl
from jax.experimental.pallas import tpu as pltpu

B, T, D = 1, 2048, 1024
H, HD, NH = 16, 64, 2
L = T * NH
EPS = 1e-5
SCALE = HD ** -0.5

PCOLS = 7168
PR_BM = 512

CT = 32
CHUNK = NH * CT
NC = T // CT

OB_M = 512

GO = 5120


def _dot(a, b):
    return jax.lax.dot_general(a, b, (((1,), (0,)), ((), ())),
                               preferred_element_type=jnp.float32)


def _dot_nt(a, b):
    return jax.lax.dot_general(a, b, (((1,), (1,)), ((), ())),
                               preferred_element_type=jnp.float32)


def _dot_tn(a, b):
    return jax.lax.dot_general(a, b, (((0,), (0,)), ((), ())),
                               preferred_element_type=jnp.float32)


def _proj_body(x_ref, wq_ref, wk_ref, wv_ref, wg_ref, wba_ref, o_ref):
    j = pl.program_id(0)

    @pl.when(j == 0)
    def _():
        o_ref[...] = _dot(x_ref[...], wq_ref[...])

    @pl.when((j == 1) | (j == 2))
    def _():
        o_ref[...] = _dot(x_ref[...], wk_ref[...])

    @pl.when((j == 3) | (j == 4))
    def _():
        o_ref[...] = _dot(x_ref[...], wv_ref[...])

    @pl.when(j == 5)
    def _():
        o_ref[...] = _dot(x_ref[...], wg_ref[...])

    @pl.when(j == 6)
    def _():
        o_ref[:, 0:128] = _dot(x_ref[...], wba_ref[...])


def _delta_body(k_ref, v_ref, q_ref, bc_ref, gc_ref, gr_ref, o_ref, s_ref):
    c = pl.program_id(0)

    @pl.when(c == 0)
    def _():
        s_ref[...] = jnp.zeros_like(s_ref)

    C = CHUNK
    row = jax.lax.broadcasted_iota(jnp.int32, (C, C), 0)
    col = jax.lax.broadcasted_iota(jnp.int32, (C, C), 1)
    incl = row >= col
    strict = row > col
    lec = row <= col
    rtok = jax.lax.broadcasted_iota(jnp.int32, (CT, C), 0)
    codd = jax.lax.broadcasted_iota(jnp.int32, (CT, C), 1)
    incl_odd = codd <= 2 * rtok + 1

    R = range(H)
    kn, qn, v, bcol, eG, eGlast, eCI, Dincl, Dstrict, S, eGo = \
        [], [], [], [], [], [], [], [], [], [], []
    for i in R:
        k = k_ref[i]
        q = q_ref[i]
        v.append(v_ref[i])
        bcol.append(bc_ref[i, 0])
        gcol = gc_ref[i, 0]
        grow = gr_ref[i, 0]
        S.append(s_ref[i])

        kn.append(k * jax.lax.rsqrt(jnp.sum(k * k, axis=1, keepdims=True) + 1e-6))
        qn.append(q * jax.lax.rsqrt(jnp.sum(q * q, axis=1, keepdims=True) + 1e-6)
                  * SCALE)

        Grow = jnp.sum(jnp.where(incl, jnp.broadcast_to(grow, (C, C)), 0.0),
                       axis=1, keepdims=True)
        Gcol = jnp.sum(jnp.where(lec, jnp.broadcast_to(gcol, (C, C)), 0.0),
                       axis=0, keepdims=True)
        Godd = jnp.sum(jnp.where(incl_odd, jnp.broadcast_to(grow, (CT, C)), 0.0),
                       axis=1, keepdims=True)
        eG.append(jnp.exp(Grow))
        eGo.append(jnp.exp(Godd))
        Glast = jnp.sum(grow)
        eGlast.append(jnp.exp(Glast))
        eCI.append(jnp.exp(Glast - Grow))
        Dfull = jnp.exp(jnp.where(incl, Grow - Gcol, -1e30))
        Dstrict.append(jnp.where(strict, Dfull, 0.0))
        Dodd = jnp.exp(jnp.where(incl_odd, Godd - Gcol, -1e30))
        Dincl.append(jnp.where(incl_odd, Dodd, 0.0))

    Np = [_dot_nt(kn[i] * bcol[i], kn[i]) * (-Dstrict[i]) for i in R]
    pred = [_dot(kn[i], S[i]) for i in R]
    attn = [_dot_nt(qn[i], kn[i]) * Dincl[i] for i in R]
    oq = [_dot(qn[i], S[i]) for i in R]
    t = [bcol[i] * (v[i] - eG[i] * pred[i]) for i in R]

    for j in range(6):
        t = [t[i] + _dot(Np[i], t[i]) for i in R]
        if j < 5:
            Np = [_dot(Np[i], Np[i]) for i in R]

    for i in R:
        o_ref[i] = eGo[i] * oq[i] + _dot(attn[i], t[i])
    for i in R:
        s_ref[i] = eGlast[i] * S[i] + _dot_tn(kn[i] * eCI[i], t[i])


def _out_body(o_ref, g_ref, w_ref, nw_ref, y_ref):
    parts = []
    for h in range(H):
        o = o_ref[h]
        on = o * jax.lax.rsqrt(jnp.mean(o * o, axis=1, keepdims=True) + EPS)
        on = on * nw_ref[...]
        gg = g_ref[:, h * HD:(h + 1) * HD]
        parts.append(on * (gg * jax.nn.sigmoid(gg)))
    y = jnp.concatenate(parts, axis=1)
    y_ref[...] = _dot(y, w_ref[...])


def kernel(x, Wq, Wk, Wv, Wb, Wa, A_log, dt_bias, Wg, norm_weight, Wo):
    x2 = x.reshape(T, D)
    Wba = jnp.pad(jnp.concatenate([Wb, Wa], axis=1), ((0, 0), (0, 80)))

    _c0 = lambda j, i: (0, 0)
    P = pl.pallas_call(
        _proj_body,
        out_shape=jax.ShapeDtypeStruct((T, PCOLS), jnp.float32),
        grid=(7, T // PR_BM),
        in_specs=[
            pl.BlockSpec((PR_BM, D), lambda j, i: (i, 0)),
            pl.BlockSpec((D, 1024), _c0),
            pl.BlockSpec((D, 1024),
                         lambda j, i: (0, jnp.clip(j - 1, 0, 1))),
            pl.BlockSpec((D, 1024),
                         lambda j, i: (0, jnp.clip(j - 3, 0, 1))),
            pl.BlockSpec((D, 1024), _c0),
            pl.BlockSpec((D, 128), _c0),
        ],
        out_specs=pl.BlockSpec((PR_BM, 1024), lambda j, i: (i, j)),
        compiler_params=pltpu.CompilerParams(
            dimension_semantics=("arbitrary", "arbitrary"),
            vmem_limit_bytes=52 * 1024 * 1024),
        name="deltanet_proj",
    )(x2, Wq, Wk, Wv, Wg, Wba)

    qraw = P[:, 0:1024]
    kraw = P[:, 1024:3072]
    vraw = P[:, 3072:5120]
    braw = P[:, 6144:6176]
    araw = P[:, 6176:6192]

    qarr = qraw.reshape(T, H, HD).transpose(1, 0, 2)
    kstep = kraw.reshape(T, NH, H, HD).transpose(2, 0, 1, 3).reshape(H, L, HD)
    vstep = vraw.reshape(T, NH, H, HD).transpose(2, 0, 1, 3).reshape(H, L, HD)

    beta = 2.0 * jax.nn.sigmoid(braw)
    beta = beta.reshape(T, NH, H).transpose(2, 0, 1).reshape(H, L)
    g_tok = -jnp.exp(A_log)[None, :] * jax.nn.softplus(araw + dt_bias[None, :])
    gstep = jnp.stack([g_tok.T, jnp.zeros((H, T), jnp.float32)], axis=2).reshape(H, L)

    b_col = beta.reshape(H, NC, CHUNK, 1)
    g_col = gstep.reshape(H, NC, CHUNK, 1)
    g_row = gstep.reshape(H, NC, 1, CHUNK)

    O_tok = pl.pallas_call(
        _delta_body,
        out_shape=jax.ShapeDtypeStruct((H, T, HD), jnp.float32),
        grid=(NC,),
        in_specs=[
            pl.BlockSpec((H, CHUNK, HD), lambda c: (0, c, 0)),
            pl.BlockSpec((H, CHUNK, HD), lambda c: (0, c, 0)),
            pl.BlockSpec((H, CT, HD), lambda c: (0, c, 0)),
            pl.BlockSpec((H, 1, CHUNK, 1), lambda c: (0, c, 0, 0)),
            pl.BlockSpec((H, 1, CHUNK, 1), lambda c: (0, c, 0, 0)),
            pl.BlockSpec((H, 1, 1, CHUNK), lambda c: (0, c, 0, 0)),
        ],
        out_specs=pl.BlockSpec((H, CT, HD), lambda c: (0, c, 0)),
        scratch_shapes=[pltpu.VMEM((H, HD, HD), jnp.float32)],
        compiler_params=pltpu.CompilerParams(
            dimension_semantics=("arbitrary",)),
        name="deltanet_chunk",
    )(kstep, vstep, qarr, b_col, g_col, g_row)

    y = pl.pallas_call(
        _out_body,
        out_shape=jax.ShapeDtypeStruct((T, D), jnp.float32),
        grid=(T // OB_M,),
        in_specs=[
            pl.BlockSpec((H, OB_M, HD), lambda i: (0, i, 0)),
            pl.BlockSpec((OB_M, H * HD), lambda i: (i, 5)),
            pl.BlockSpec((H * HD, D), lambda i: (0, 0)),
            pl.BlockSpec((1, HD), lambda i: (0, 0)),
        ],
        out_specs=pl.BlockSpec((OB_M, D), lambda i: (i, 0)),
        compiler_params=pltpu.CompilerParams(
            dimension_semantics=("arbitrary",)),
        name="deltanet_out",
    )(O_tok, P, Wo, norm_weight.reshape(1, HD))

    return y.reshape(B, T, D)
